# Initial kernel scaffold; baseline (speedup 1.0000x reference)
import jax
import jax.numpy as jnp
from jax.experimental import pallas as pl


def kernel(entity_emb, relation_emb, edge_index, edge_type):
    raise NotImplementedError("write your pallas kernel here")


# speedup vs baseline: 2.6471x; 2.6471x over previous
import functools

import jax
import jax.numpy as jnp
from jax import lax
from jax.experimental import pallas as pl
from jax.experimental.pallas import tpu as pltpu
from jax.experimental.pallas import tpu_sc as plsc

N_ENT = 10000
N_REL = 24
D = 256
E = 160000
RES_LAMBDA = 0.5
MIN_NORM = 1e-15
MAX_NORM = 1.0 - 1e-5

W = 128
IDX_ROWS = E // W
GIDX_ROWS = 2 * E // W
NC, NS = 2, 16
NW = NC * NS
HALF = D // 2
CHUNK = 624
TAIL = N_ENT - NS * CHUNK
BE = 2000
NB = E // BE
RB = 1000

@functools.lru_cache(maxsize=None)
def _get_sc_gather():
    mesh = plsc.VectorSubcoreMesh(
        core_axis_name="c", subcore_axis_name="s",
        num_cores=NC, num_subcores=NS)
    return pl.kernel(
        _sc_gather_body,
        mesh=mesh,
        out_type=jax.ShapeDtypeStruct((2 * E, D), jnp.float32),
        scratch_types=[
            pltpu.VMEM((W,), jnp.int32),
            pltpu.VMEM((W, D), jnp.float32),
            pltpu.SemaphoreType.DMA,
        ],
    )


def _sc_gather_body(table_hbm, idx_hbm, out_hbm, idx_v, rows_v, sem):
    wid = lax.axis_index("s") * NC + lax.axis_index("c")

    @pl.loop(0, (GIDX_ROWS + NW - 1) // NW)
    def _(w):
        r = wid + NW * w

        @pl.when(r < GIDX_ROWS)
        def _():
            pltpu.sync_copy(idx_hbm.at[pl.ds(r * W, W)], idx_v)
            pltpu.async_copy(table_hbm.at[idx_v], rows_v, sem).wait()
            pltpu.sync_copy(rows_v, out_hbm.at[pl.ds(r * W, W)])


@functools.lru_cache(maxsize=None)
def _get_sc_scatter():
    mesh = plsc.VectorSubcoreMesh(
        core_axis_name="c", subcore_axis_name="s",
        num_cores=NC, num_subcores=NS)
    return pl.kernel(
        _sc_scatter_body,
        mesh=mesh,
        out_type=[
            jax.ShapeDtypeStruct((N_ENT, HALF), jnp.float32),
            jax.ShapeDtypeStruct((N_ENT, HALF), jnp.float32),
        ],
        scratch_types=[
            pltpu.VMEM((W,), jnp.int32),
            pltpu.VMEM((W, HALF), jnp.float32),
            pltpu.VMEM_SHARED((N_ENT, HALF), jnp.float32),
        ],
    )


def _sc_scatter_body(idx_hbm, res_lo_hbm, res_hi_hbm, zeros_hbm,
                     out_lo_hbm, out_hi_hbm, idx_v, buf_v, acc_sh):
    c = lax.axis_index("c")
    s = lax.axis_index("s")

    pltpu.sync_copy(zeros_hbm, acc_sh.at[pl.ds(s * CHUNK, CHUNK)])

    @pl.when(s == 0)
    def _():
        pltpu.sync_copy(zeros_hbm.at[pl.ds(0, TAIL)],
                        acc_sh.at[pl.ds(NS * CHUNK, TAIL)])

    plsc.subcore_barrier()

    def accumulate(res_hbm):
        @pl.loop(0, (IDX_ROWS + NS - 1) // NS)
        def _(w):
            r = s + NS * w

            @pl.when(r < IDX_ROWS)
            def _():
                pltpu.sync_copy(idx_hbm.at[pl.ds(r * W, W)], idx_v)
                pltpu.sync_copy(res_hbm.at[pl.ds(r * W, W)], buf_v)
                pltpu.sync_copy(buf_v, acc_sh.at[idx_v], add=True)

    @pl.when(c == 0)
    def _():
        accumulate(res_lo_hbm)

    @pl.when(c == 1)
    def _():
        accumulate(res_hi_hbm)

    plsc.subcore_barrier()

    def writeout(out_hbm):
        pltpu.sync_copy(acc_sh.at[pl.ds(s * CHUNK, CHUNK)],
                        out_hbm.at[pl.ds(s * CHUNK, CHUNK)])

        @pl.when(s == 0)
        def _():
            pltpu.sync_copy(acc_sh.at[pl.ds(NS * CHUNK, TAIL)],
                            out_hbm.at[pl.ds(NS * CHUNK, TAIL)])

    @pl.when(c == 0)
    def _():
        writeout(out_lo_hbm)

    @pl.when(c == 1)
    def _():
        writeout(out_hi_hbm)


def _nrm(x):
    return jnp.sqrt(jnp.clip(jnp.sum(x * x, axis=-1, keepdims=True), MIN_NORM, None))


def _project(x):
    n = _nrm(x)
    return jnp.where(n > MAX_NORM, x / n * MAX_NORM, x)


def _mobius_add(x, y):
    x2 = jnp.sum(x * x, axis=-1, keepdims=True)
    y2 = jnp.sum(y * y, axis=-1, keepdims=True)
    xy = jnp.sum(x * y, axis=-1, keepdims=True)
    num = (1.0 + 2.0 * xy + y2) * x + (1.0 - x2) * y
    den = 1.0 + 2.0 * xy + x2 * y2
    return num / jnp.clip(den, MIN_NORM, None)


def _edge_math_body(gh_ref, gt_ref, rt_ref, rel_ref, lo_ref, hi_ref):
    h = gh_ref[...]
    t = gt_ref[...]
    rt = rt_ref[0, 0, :] - 1
    onehot = (lax.broadcasted_iota(jnp.int32, (BE, N_REL), 1)
              == rt[:, None]).astype(jnp.float32)
    r = jnp.dot(onehot, rel_ref[...], preferred_element_type=jnp.float32)

    nh = _nrm(h)
    p = _project(jnp.tanh(nh) * h / nh)

    lam = 2.0 / jnp.clip(1.0 - jnp.sum(p * p, axis=-1, keepdims=True), MIN_NORM, None)

    def expmap_p(u):
        n = _nrm(u)
        second = jnp.tanh(0.5 * lam * n) * u / n
        return _project(_mobius_add(p, second))

    hyper_tail = expmap_p(t)
    hyper_rel = expmap_p(r)
    res = _project(_mobius_add(hyper_tail, hyper_rel))

    sub = _mobius_add(-p, res)
    ns = _nrm(sub)
    nc = jnp.clip(ns, -1.0 + 1e-7, 1.0 - 1e-7)
    artanh = 0.5 * jnp.log((1.0 + nc) / (1.0 - nc))
    res = (2.0 / lam) * artanh * sub / ns

    ricci = t + r
    rn = ricci / jnp.clip(
        jnp.sqrt(jnp.sum(ricci * ricci, axis=-1, keepdims=True)), 1e-12, None)
    res = jax.nn.relu(res + rn * 1e-7)

    lo_ref[...] = res[:, :HALF]
    hi_ref[...] = res[:, HALF:]


def _edge_math(gathered, rtype3, relation_emb):
    return pl.pallas_call(
        _edge_math_body,
        grid=(NB,),
        in_specs=[
            pl.BlockSpec((BE, D), lambda i: (i, 0)),
            pl.BlockSpec((BE, D), lambda i: (i + NB, 0)),
            pl.BlockSpec((1, 1, BE), lambda i: (i, 0, 0)),
            pl.BlockSpec((N_REL, D), lambda i: (0, 0)),
        ],
        out_specs=[
            pl.BlockSpec((BE, HALF), lambda i: (i, 0)),
            pl.BlockSpec((BE, HALF), lambda i: (i, 0)),
        ],
        out_shape=[
            jax.ShapeDtypeStruct((E, HALF), jnp.float32),
            jax.ShapeDtypeStruct((E, HALF), jnp.float32),
        ],
    )(gathered, gathered, rtype3, relation_emb)


def _norm_body(lo_ref, hi_ref, out_ref):
    sums = jnp.concatenate([lo_ref[...], hi_ref[...]], axis=-1)
    n = jnp.sqrt(jnp.sum(sums * sums, axis=-1, keepdims=True))
    out_ref[...] = sums / jnp.clip(n, 1e-12, None)


def _normalize(lo, hi):
    return pl.pallas_call(
        _norm_body,
        grid=(N_ENT // RB,),
        in_specs=[
            pl.BlockSpec((RB, HALF), lambda i: (i, 0)),
            pl.BlockSpec((RB, HALF), lambda i: (i, 0)),
        ],
        out_specs=pl.BlockSpec((RB, D), lambda i: (i, 0)),
        out_shape=jax.ShapeDtypeStruct((N_ENT, D), jnp.float32),
    )(lo, hi)


def _final_body(lo_ref, hi_ref, n1_ref, e0_ref, out_ref):
    sums = jnp.concatenate([lo_ref[...], hi_ref[...]], axis=-1)
    n = jnp.sqrt(jnp.sum(sums * sums, axis=-1, keepdims=True))
    ent2 = sums / jnp.clip(n, 1e-12, None)
    out_ref[...] = (RES_LAMBDA * RES_LAMBDA) * e0_ref[...] + RES_LAMBDA * n1_ref[...] + ent2


def _final(lo, hi, n1, e0):
    return pl.pallas_call(
        _final_body,
        grid=(N_ENT // RB,),
        in_specs=[
            pl.BlockSpec((RB, HALF), lambda i: (i, 0)),
            pl.BlockSpec((RB, HALF), lambda i: (i, 0)),
            pl.BlockSpec((RB, D), lambda i: (i, 0)),
            pl.BlockSpec((RB, D), lambda i: (i, 0)),
        ],
        out_specs=pl.BlockSpec((RB, D), lambda i: (i, 0)),
        out_shape=jax.ShapeDtypeStruct((N_ENT, D), jnp.float32),
    )(lo, hi, n1, e0)


def kernel(entity_emb, relation_emb, edge_index, edge_type):
    head = edge_index[0]
    tail = edge_index[1]
    gidx = jnp.concatenate([head, tail])
    sidx = head
    rtype3 = edge_type.reshape(NB, 1, BE)
    zeros = jnp.zeros((CHUNK, HALF), jnp.float32)

    sc_gather = _get_sc_gather()
    sc_scatter = _get_sc_scatter()

    def hop(ent):
        gathered = sc_gather(ent, gidx)
        res_lo, res_hi = _edge_math(gathered, rtype3, relation_emb)
        return sc_scatter(sidx, res_lo, res_hi, zeros)

    lo1, hi1 = hop(entity_emb)
    n1 = _normalize(lo1, hi1)
    lo2, hi2 = hop(n1)
    return _final(lo2, hi2, n1, entity_emb)
